# baseline (speedup 1.0000x reference)
import jax
import jax.numpy as jnp
from jax import lax
from jax.experimental import pallas as pl
from jax.experimental.pallas import tpu as pltpu
from jax.experimental.pallas import tpu_sc as plsc

_K = 5
_BATCH = 16384
_C = 1000
_NW = 32
_RPW = _BATCH // _NW
_RB = 32
_NBAT = _RPW // _RB
_STRIDE = 1008
_NCH = _STRIDE // 16


def _sort(v, desc):
    s, _ = plsc.sort_key_val(v, v, descending=desc)
    return s


def _sc_body(prob_hbm, lab_hbm, out_hbm, buf0, buf1, labv, outv, sem0, sem1):
    cid = lax.axis_index("c")
    sid = lax.axis_index("s")
    wid = sid * 2 + cid
    row0 = wid * _RPW

    neg = jnp.full((16,), -jnp.inf, jnp.float32)
    lane = lax.broadcasted_iota(jnp.int32, (16,), 0)
    five = jnp.full((16,), _K, jnp.int32)
    ones = jnp.ones((16,), jnp.int32)
    zeros = jnp.zeros((16,), jnp.int32)

    for r in range(_RB):
        buf0[r, pl.ds(992, 16)] = neg
        buf1[r, pl.ds(992, 16)] = neg

    pltpu.sync_copy(lab_hbm.at[pl.ds(row0, _RPW)], labv)

    def _issue(b, buf, sem):
        rbase = row0 + b * _RB
        for r in range(_RB):
            pltpu.async_copy(prob_hbm.at[rbase + r, :],
                             buf.at[r, pl.ds(0, _C)], sem)

    def _drain(b, buf, sem):
        rbase = row0 + b * _RB
        for r in range(_RB):
            pltpu.make_async_copy(prob_hbm.at[rbase + r, :],
                                  buf.at[r, pl.ds(0, _C)], sem).wait()

    def _top16(buf, r):
        def build(lo, width, desc):
            if lo >= _NCH:
                return neg
            if width == 1:
                return _sort(buf[r, pl.ds(lo * 16, 16)], desc)
            if lo + width // 2 >= _NCH:
                return build(lo, width // 2, desc)
            a = build(lo, width // 2, False)
            b = build(lo + width // 2, width // 2, True)
            return _sort(jnp.maximum(a, b), desc)

        return build(0, 64, True)

    def _compute(buf, b, iou_acc):
        def group_body(g, iou_acc):
            def row_body(j, carry):
                thr_vec, cnt_vec = carry
                r = g * 16 + j
                acc = _top16(buf, r)
                thr = lax.gather(
                    acc, five[:, None],
                    lax.GatherDimensionNumbers(
                        offset_dims=(), collapsed_slice_dims=(0,),
                        start_index_map=(0,)),
                    slice_sizes=(1,),
                    mode=lax.GatherScatterMode.PROMISE_IN_BOUNDS)
                cntv = plsc.all_reduce_population_count(acc > thr)
                sel = lane == j
                return (jnp.where(sel, thr, thr_vec),
                        jnp.where(sel, cntv, cnt_vec))

            thr_vec, cnt_vec = lax.fori_loop(
                0, 16, row_body,
                (jnp.zeros((16,), jnp.float32), jnp.zeros((16,), jnp.int32)))

            rows16 = g * 16 + lane
            lab16 = labv[pl.ds(b * _RB + g * 16, 16)]
            labval = plsc.load_gather(buf, [rows16, lab16])
            inter = jnp.where(labval > thr_vec, ones, zeros)
            union = cnt_vec + ones - inter
            iou = inter.astype(jnp.float32) / union.astype(jnp.float32)
            return iou_acc + iou

        return lax.fori_loop(0, _RB // 16, group_body, iou_acc)

    _issue(0, buf0, sem0)

    def super_body(i, iou_acc):
        b0 = 2 * i
        _issue(b0 + 1, buf1, sem1)
        _drain(b0, buf0, sem0)
        iou_acc = _compute(buf0, b0, iou_acc)

        @pl.when(i < _NBAT // 2 - 1)
        def _():
            _issue(b0 + 2, buf0, sem0)

        _drain(b0 + 1, buf1, sem1)
        return _compute(buf1, b0 + 1, iou_acc)

    iou_acc = lax.fori_loop(0, _NBAT // 2, super_body,
                            jnp.zeros((16,), jnp.float32))
    outv[...] = iou_acc
    pltpu.sync_copy(outv, out_hbm.at[wid])


@jax.jit
def kernel(prob, label):
    mesh = plsc.VectorSubcoreMesh(core_axis_name="c", subcore_axis_name="s")
    out = pl.kernel(
        _sc_body,
        out_type=jax.ShapeDtypeStruct((_NW, 16), jnp.float32),
        mesh=mesh,
        scratch_types=[
            pltpu.VMEM((_RB, _STRIDE), jnp.float32),
            pltpu.VMEM((_RB, _STRIDE), jnp.float32),
            pltpu.VMEM((_RPW,), jnp.int32),
            pltpu.VMEM((16,), jnp.float32),
            pltpu.SemaphoreType.DMA,
            pltpu.SemaphoreType.DMA,
        ],
        compiler_params=pltpu.CompilerParams(use_tc_tiling_on_sc=False,
                                             needs_layout_passes=False),
    )(prob, label)
    return jnp.sum(out) / jnp.float32(_BATCH)


# speedup vs baseline: 1.0017x; 1.0017x over previous
import jax
import jax.numpy as jnp
from jax import lax
from jax.experimental import pallas as pl
from jax.experimental.pallas import tpu as pltpu
from jax.experimental.pallas import tpu_sc as plsc

_K = 5
_BATCH = 16384
_C = 1000
_NW = 32
_RPW = _BATCH // _NW
_RB = 32
_NBAT = _RPW // _RB
_STRIDE = 1008
_NCH = _STRIDE // 16


def _sort(v, desc):
    s, _ = plsc.sort_key_val(v, v, descending=desc)
    return s


def _sc_body(prob_hbm, lab_hbm, out_hbm, buf0, buf1, labv, outv, sem0, sem1):
    cid = lax.axis_index("c")
    sid = lax.axis_index("s")
    wid = sid * 2 + cid
    row0 = wid * _RPW

    neg = jnp.full((16,), -jnp.inf, jnp.float32)
    lane = lax.broadcasted_iota(jnp.int32, (16,), 0)
    five = jnp.full((16,), _K, jnp.int32)
    ones = jnp.ones((16,), jnp.int32)
    zeros = jnp.zeros((16,), jnp.int32)

    for r in range(_RB):
        buf0[r, pl.ds(992, 16)] = neg
        buf1[r, pl.ds(992, 16)] = neg

    pltpu.sync_copy(lab_hbm.at[pl.ds(row0, _RPW)], labv)

    def _issue(b, buf, sem):
        rbase = row0 + b * _RB
        for r in range(_RB):
            pltpu.async_copy(prob_hbm.at[pl.ds((rbase + r) * _C, _C)],
                             buf.at[r, pl.ds(0, _C)], sem)

    def _drain(b, buf, sem):
        rbase = row0 + b * _RB
        for r in range(_RB):
            pltpu.make_async_copy(prob_hbm.at[pl.ds((rbase + r) * _C, _C)],
                                  buf.at[r, pl.ds(0, _C)], sem).wait()

    def _top16(buf, r):
        def build(lo, width, desc):
            if lo >= _NCH:
                return neg
            if width == 1:
                return _sort(buf[r, pl.ds(lo * 16, 16)], desc)
            if lo + width // 2 >= _NCH:
                return build(lo, width // 2, desc)
            a = build(lo, width // 2, False)
            b = build(lo + width // 2, width // 2, True)
            return _sort(jnp.maximum(a, b), desc)

        return build(0, 64, True)

    def _compute(buf, b, iou_acc):
        def group_body(g, iou_acc):
            def row_body(j, carry):
                thr_vec, cnt_vec = carry
                r = g * 16 + j
                acc = _top16(buf, r)
                thr = lax.gather(
                    acc, five[:, None],
                    lax.GatherDimensionNumbers(
                        offset_dims=(), collapsed_slice_dims=(0,),
                        start_index_map=(0,)),
                    slice_sizes=(1,),
                    mode=lax.GatherScatterMode.PROMISE_IN_BOUNDS)
                cntv = plsc.all_reduce_population_count(acc > thr)
                sel = lane == j
                return (jnp.where(sel, thr, thr_vec),
                        jnp.where(sel, cntv, cnt_vec))

            thr_vec, cnt_vec = lax.fori_loop(
                0, 16, row_body,
                (jnp.zeros((16,), jnp.float32), jnp.zeros((16,), jnp.int32)))

            rows16 = g * 16 + lane
            lab16 = labv[pl.ds(b * _RB + g * 16, 16)]
            labval = plsc.load_gather(buf, [rows16, lab16])
            inter = jnp.where(labval > thr_vec, ones, zeros)
            union = cnt_vec + ones - inter
            iou = inter.astype(jnp.float32) / union.astype(jnp.float32)
            return iou_acc + iou

        return lax.fori_loop(0, _RB // 16, group_body, iou_acc)

    _issue(0, buf0, sem0)

    def super_body(i, iou_acc):
        b0 = 2 * i
        _issue(b0 + 1, buf1, sem1)
        _drain(b0, buf0, sem0)
        iou_acc = _compute(buf0, b0, iou_acc)

        @pl.when(i < _NBAT // 2 - 1)
        def _():
            _issue(b0 + 2, buf0, sem0)

        _drain(b0 + 1, buf1, sem1)
        return _compute(buf1, b0 + 1, iou_acc)

    iou_acc = lax.fori_loop(0, _NBAT // 2, super_body,
                            jnp.zeros((16,), jnp.float32))
    outv[...] = iou_acc
    pltpu.sync_copy(outv, out_hbm.at[wid])


@jax.jit
def kernel(prob, label):
    mesh = plsc.VectorSubcoreMesh(core_axis_name="c", subcore_axis_name="s")
    out = pl.kernel(
        _sc_body,
        out_type=jax.ShapeDtypeStruct((_NW, 16), jnp.float32),
        mesh=mesh,
        scratch_types=[
            pltpu.VMEM((_RB, _STRIDE), jnp.float32),
            pltpu.VMEM((_RB, _STRIDE), jnp.float32),
            pltpu.VMEM((_RPW,), jnp.int32),
            pltpu.VMEM((16,), jnp.float32),
            pltpu.SemaphoreType.DMA,
            pltpu.SemaphoreType.DMA,
        ],
        compiler_params=pltpu.CompilerParams(use_tc_tiling_on_sc=False,
                                             needs_layout_passes=False),
    )(prob.reshape(-1), label)
    return jnp.sum(out) / jnp.float32(_BATCH)
